# baseline (speedup 1.0000x reference)
import functools

import jax
import jax.numpy as jnp
from jax import lax
from jax.experimental import pallas as pl
from jax.experimental.pallas import tpu as pltpu
from jax.experimental.pallas import tpu_sc as plsc

N = 9
EM = 36
NFD = 11
LAT = 128
HH = N * NFD
ODIM = N * (N + 1) // 2
NEG = -1e30
F32 = jnp.float32


def _body1(adj_ref, ef_ref, nf_ref, Wmu_ref, bmu_ref, Wls_ref, bls_ref,
           Wd1_ref, bd1_ref, Wd2_ref, bd2_ref, Wnd_ref, bnd_ref,
           Wed_ref, bed_ref, eps_ref,
           x_ref, adj_ref16, logp_ref, log1mp_ref, part_ref):
    adj = adj_ref[...]
    ef_all = ef_ref[...]
    gh = nf_ref[...]
    eps = eps_ref[...]

    dot = functools.partial(jnp.dot, preferred_element_type=jnp.float32)
    z_mu = dot(gh, Wmu_ref[...]) + bmu_ref[...]
    z_ls = dot(gh, Wls_ref[...]) + bls_ref[...]
    z = z_mu + eps * jnp.exp(0.5 * z_ls)
    y = jnp.maximum(dot(z, Wd1_ref[...]) + bd1_ref[...], 0.0)
    hdec = dot(y, Wd2_ref[...]) + bd2_ref[...]
    out = jax.nn.sigmoid(hdec)
    node_recon = dot(y, Wnd_ref[...]) + bnd_ref[...]
    ed144 = dot(y, Wed_ref[...]) + bed_ref[...]
    e_r = lax.broadcasted_iota(jnp.int32, (EM, 4 * EM), 0)
    e_c = lax.broadcasted_iota(jnp.int32, (EM, 4 * EM), 1)
    Asel = ((e_c // 4) == e_r).astype(F32)
    b_r = lax.broadcasted_iota(jnp.int32, (4 * EM, 4), 0)
    b_c = lax.broadcasted_iota(jnp.int32, (4 * EM, 4), 1)
    Bsel = ((b_r % 4) == b_c).astype(F32)
    ed_logits = dot(Asel * ed144, Bsel)

    edm = jnp.max(ed_logits, axis=1, keepdims=True)
    ede = jnp.exp(ed_logits - edm)
    er = ede / jnp.sum(ede, axis=1, keepdims=True)

    rows = []
    base = 0
    for r in range(N):
        seg = out[:, base:base + (N - r)]
        if r > 0:
            seg = jnp.concatenate([jnp.zeros((1, r), F32), seg], axis=1)
        rows.append(seg)
        base += N - r
    low = jnp.concatenate(rows, axis=0)

    r9 = lax.broadcasted_iota(jnp.int32, (N, N), 0)
    c9 = lax.broadcasted_iota(jnp.int32, (N, N), 1)
    eyeM = (r9 == c9).astype(F32)

    def _tr(m):
        return lax.dot_general(eyeM, m, (((1,), (1,)), ((), ())),
                               preferred_element_type=jnp.float32)

    lowT = _tr(low)
    adjr = low + lowT - low * eyeM

    adjT = _tr(adj)
    aw_col = jnp.concatenate(
        [adjT[r + 1:N, r:r + 1] for r in range(N - 1)], axis=0)
    edges_total = er * aw_col

    ef9 = ef_all[:N, :]
    efr9 = er[:N, :]
    outer = functools.partial(
        lax.dot_general, dimension_numbers=(((1,), (1,)), ((), ())),
        preferred_element_type=jnp.float32)
    dots = outer(ef9, efr9)
    n1 = jnp.sqrt(jnp.sum(ef9 * ef9, axis=1, keepdims=True))
    n2 = jnp.sqrt(jnp.sum(efr9 * efr9, axis=1, keepdims=True))
    denom = jnp.maximum(outer(n1, n2), 1e-8)
    cosm = dots / denom

    dadj = jnp.sum(adj * eyeM, axis=1, keepdims=True)
    dadjr = jnp.sum(adjr * eyeM, axis=1, keepdims=True)
    diag_term = outer(dadj, dadjr) * cosm

    f_r = lax.broadcasted_iota(jnp.int32, (N * N, N), 0)
    f_c = lax.broadcasted_iota(jnp.int32, (N * N, N), 1)
    RowSel = ((f_r // N) == f_c).astype(F32)
    ModMsk = ((f_r % N) == f_c).astype(F32)
    adj_col = jnp.sum(dot(RowSel, adj) * ModMsk,
                      axis=1, keepdims=True)

    g_r = lax.broadcasted_iota(jnp.int32, (N, N * N), 0)
    g_c = lax.broadcasted_iota(jnp.int32, (N, N * N), 1)
    ColSel = (g_r == (g_c % N)).astype(F32)
    DivMsk = (g_r == (g_c // N)).astype(F32)
    adjr_row = jnp.sum(dot(adjr, ColSel) * DivMsk,
                       axis=0, keepdims=True)
    base_S = jnp.abs(adj_col - adjr_row)

    vR = lax.broadcasted_iota(jnp.int32, (N * N, 1), 0)
    vC = lax.broadcasted_iota(jnp.int32, (1, N * N), 1)
    eyeR = (vR // N) == (vR % N)
    eyeC = (vC // N) == (vC % N)
    offmask = ((~eyeR) & (~eyeC)).astype(F32)

    dt_c = jnp.concatenate([diag_term] * N, axis=1)
    dt_tile = jnp.concatenate([dt_c] * N, axis=0)
    S = jnp.where(eyeR & eyeC, dt_tile, base_S * offmask)

    rm_r = lax.broadcasted_iota(jnp.int32, (N, N * N), 0)
    rm_c = lax.broadcasted_iota(jnp.int32, (N, N * N), 1)
    Rm = (((rm_c // N) == rm_r) & ((rm_c % N) != rm_r)).astype(F32)

    def mpm_core(x):
        xcols = jnp.concatenate([x] * N, axis=0)
        pmax = jnp.concatenate(
            [jnp.max(S[:, a * N:(a + 1) * N] * xcols, axis=1, keepdims=True)
             for a in range(N)], axis=1)
        neigh = dot(Rm, pmax)
        return x * diag_term + neigh

    def mpm_outer(_, x):
        for _step in range(10):
            x = mpm_core(x)
        return x / jnp.sqrt(jnp.sum(x * x))

    x0 = jnp.full((N, N), 1.0 / N, F32)
    x = lax.fori_loop(0, 5, mpm_outer, x0)

    loss_kl = -0.5 * jnp.sum(1.0 + z_ls - z_mu * z_mu - jnp.exp(z_ls)) / (N * N)
    diff_e = edges_total - ef_all
    loss_edge = jnp.sum(diff_e * diff_e) / (EM * 4)
    diff_n = node_recon - gh
    loss_node = jnp.sum(diff_n * diff_n) / HH

    pclip = jnp.clip(low, 1e-7, 1.0 - 1e-7)
    logp = jnp.log(pclip)
    log1mp = jnp.log(1.0 - pclip)

    pad7 = jnp.zeros((N, 16 - N), F32)
    x_ref[...] = jnp.concatenate([x, pad7], axis=1)
    adj_ref16[...] = jnp.concatenate([adj, pad7], axis=1)
    logp_ref[...] = jnp.concatenate([logp, pad7], axis=1)
    log1mp_ref[...] = jnp.concatenate([log1mp, pad7], axis=1)
    part_ref[...] = jnp.broadcast_to(loss_kl + loss_edge + loss_node, (1, 16))


_SC_MESH = plsc.VectorSubcoreMesh(core_axis_name="c", subcore_axis_name="s")


@functools.partial(
    pl.kernel, mesh=_SC_MESH,
    compiler_params=pltpu.CompilerParams(use_tc_tiling_on_sc=False,
                                         needs_layout_passes=False),
    out_type=jax.ShapeDtypeStruct((1, 16), jnp.float32),
    scratch_types=[
        pltpu.VMEM((N, 16), jnp.float32),
        pltpu.VMEM((N * 16,), jnp.float32),
        pltpu.VMEM((N, 16), jnp.float32),
        pltpu.VMEM((N, 16), jnp.float32),
        pltpu.VMEM((1, 16), jnp.float32),
        pltpu.VMEM((512,), jnp.float32),
        pltpu.VMEM((512,), jnp.float32),
        pltpu.VMEM((9 * 512,), jnp.int32),
        pltpu.VMEM((1, 16), jnp.float32),
    ])
def _sc_assign(x_hbm, adj_hbm, logp_hbm, log1mp_hbm, part_hbm, out_hbm,
               x_v, adj_v, logp_v, log1mp_v, part_v, g_a, g_b, am_v,
               res_v):
    @pl.when((lax.axis_index("c") == 0) & (lax.axis_index("s") == 0))
    def _():
        pltpu.sync_copy(x_hbm, x_v)
        pltpu.sync_copy(adj_hbm, adj_v)
        pltpu.sync_copy(logp_hbm, logp_v)
        pltpu.sync_copy(log1mp_hbm, log1mp_v)
        pltpu.sync_copy(part_hbm, part_v)
        iota16 = lax.iota(jnp.int32, 16)
        neg = jnp.full((16,), NEG, jnp.float32)
        zero16 = jnp.zeros((16,), jnp.float32)

        def zinit(k, c):
            g_a[pl.ds(k * 16, 16)] = zero16
            return c
        lax.fori_loop(0, 32, zinit, 0)

        bufs = [g_a, g_b]
        for i in range(N - 1, -1, -1):
            gp = bufs[(8 - i) % 2]
            gn = bufs[(9 - i) % 2]

            xrow = x_v[i, :]

            def round_body(k, c, i=i, gp=gp, gn=gn, xrow=xrow):
                base = k * 16
                svec = base + iota16
                m = neg
                am = jnp.zeros((16,), jnp.int32)
                for j in range(N):
                    bit = ((svec >> j) & 1) == 1
                    gat = plsc.load_gather(gp, [svec - (1 << j)], mask=bit)
                    cand = jnp.where(bit, gat + xrow[j], neg)
                    am = jnp.where(cand > m, j, am)
                    m = jnp.maximum(m, cand)
                gn[pl.ds(base, 16)] = m
                am_v[pl.ds(i * 512 + base, 16)] = am
                return c
            lax.fori_loop(0, 32, round_body, 0)

        scur = jnp.full((16,), 511, jnp.int32)
        ind = jnp.zeros((16,), jnp.int32)
        jp_list = []
        for i in range(N):
            jp = plsc.load_gather(am_v, [i * 512 + scur])
            jp_list.append(jp)
            ind = ind + jnp.where(iota16 == jp, i, 0)
            scur = scur - (jnp.int32(1) << jp)

        acc = jnp.zeros((16,), jnp.float32)
        for r in range(N):
            ind_r = jnp.zeros((16,), jnp.int32)
            for i in range(N):
                ind_r = ind_r + jnp.where(jp_list[i] == r, i, 0)
            a_row = plsc.load_gather(adj_v, [ind_r * 16 + ind])
            lp = logp_v[r, :]
            l1 = log1mp_v[r, :]
            term = a_row * lp + (1.0 - a_row) * l1
            mask = (iota16 >= r) & (iota16 < N)
            acc = acc + jnp.where(mask, term, 0.0)
        s16 = jnp.broadcast_to(jnp.sum(acc), (16,))
        total = part_v[0, :] - s16 / jnp.full((16,), float(ODIM), jnp.float32)
        res_v[0, :] = total
        pltpu.sync_copy(res_v, out_hbm)


def kernel(adj, edges_features, nodes_features, W_mu, b_mu, W_ls, b_ls,
           W_d1, b_d1, W_d2, b_d2, W_nd, b_nd, W_ed, b_ed, eps):
    adj0 = adj[0]
    ef = edges_features[0]
    gh = nodes_features.reshape(1, HH)
    x16, adj16, logp16, log1mp16, part = pl.pallas_call(
        _body1,
        out_shape=[jax.ShapeDtypeStruct((N, 16), jnp.float32),
                   jax.ShapeDtypeStruct((N, 16), jnp.float32),
                   jax.ShapeDtypeStruct((N, 16), jnp.float32),
                   jax.ShapeDtypeStruct((N, 16), jnp.float32),
                   jax.ShapeDtypeStruct((1, 16), jnp.float32)],
    )(adj0, ef, gh,
      W_mu, b_mu.reshape(1, -1), W_ls, b_ls.reshape(1, -1),
      W_d1, b_d1.reshape(1, -1), W_d2, b_d2.reshape(1, -1),
      W_nd, b_nd.reshape(1, -1), W_ed, b_ed.reshape(1, -1),
      eps.reshape(1, -1))

    res = _sc_assign(x16, adj16.reshape(N * 16), logp16, log1mp16, part)
    return res[0, 0]


# speedup vs baseline: 1.2465x; 1.2465x over previous
import functools

import jax
import jax.numpy as jnp
from jax import lax
from jax.experimental import pallas as pl
from jax.experimental.pallas import tpu as pltpu
from jax.experimental.pallas import tpu_sc as plsc

N = 9
EM = 36
NFD = 11
LAT = 128
HH = N * NFD
ODIM = N * (N + 1) // 2
NEG = -1e30
F32 = jnp.float32


def _body1(adj_ref, ef_ref, nf_ref, Wmu_ref, bmu_ref, Wls_ref, bls_ref,
           Wd1_ref, bd1_ref, Wd2_ref, bd2_ref, Wnd_ref, bnd_ref,
           Wed_ref, bed_ref, eps_ref,
           x_ref, adj_ref16, logp_ref, log1mp_ref, part_ref):
    adj = adj_ref[...]
    ef_all = ef_ref[...]
    gh = nf_ref[...]
    eps = eps_ref[...]

    dot = functools.partial(jnp.dot, preferred_element_type=jnp.float32)
    z_mu = dot(gh, Wmu_ref[...]) + bmu_ref[...]
    z_ls = dot(gh, Wls_ref[...]) + bls_ref[...]
    z = z_mu + eps * jnp.exp(0.5 * z_ls)
    y = jnp.maximum(dot(z, Wd1_ref[...]) + bd1_ref[...], 0.0)
    hdec = dot(y, Wd2_ref[...]) + bd2_ref[...]
    out = jax.nn.sigmoid(hdec)
    node_recon = dot(y, Wnd_ref[...]) + bnd_ref[...]
    ed144 = dot(y, Wed_ref[...]) + bed_ref[...]
    e_r = lax.broadcasted_iota(jnp.int32, (EM, 4 * EM), 0)
    e_c = lax.broadcasted_iota(jnp.int32, (EM, 4 * EM), 1)
    Asel = ((e_c // 4) == e_r).astype(F32)
    b_r = lax.broadcasted_iota(jnp.int32, (4 * EM, 4), 0)
    b_c = lax.broadcasted_iota(jnp.int32, (4 * EM, 4), 1)
    Bsel = ((b_r % 4) == b_c).astype(F32)
    ed_logits = dot(Asel * ed144, Bsel)

    edm = jnp.max(ed_logits, axis=1, keepdims=True)
    ede = jnp.exp(ed_logits - edm)
    er = ede / jnp.sum(ede, axis=1, keepdims=True)

    rows = []
    base = 0
    for r in range(N):
        seg = out[:, base:base + (N - r)]
        if r > 0:
            seg = jnp.concatenate([jnp.zeros((1, r), F32), seg], axis=1)
        rows.append(seg)
        base += N - r
    low = jnp.concatenate(rows, axis=0)

    r9 = lax.broadcasted_iota(jnp.int32, (N, N), 0)
    c9 = lax.broadcasted_iota(jnp.int32, (N, N), 1)
    eyeM = (r9 == c9).astype(F32)

    def _tr(m):
        return lax.dot_general(eyeM, m, (((1,), (1,)), ((), ())),
                               preferred_element_type=jnp.float32)

    lowT = _tr(low)
    adjr = low + lowT - low * eyeM

    adjT = _tr(adj)
    aw_col = jnp.concatenate(
        [adjT[r + 1:N, r:r + 1] for r in range(N - 1)], axis=0)
    edges_total = er * aw_col

    ef9 = ef_all[:N, :]
    efr9 = er[:N, :]
    outer = functools.partial(
        lax.dot_general, dimension_numbers=(((1,), (1,)), ((), ())),
        preferred_element_type=jnp.float32)
    dots = outer(ef9, efr9)
    n1 = jnp.sqrt(jnp.sum(ef9 * ef9, axis=1, keepdims=True))
    n2 = jnp.sqrt(jnp.sum(efr9 * efr9, axis=1, keepdims=True))
    denom = jnp.maximum(outer(n1, n2), 1e-8)
    cosm = dots / denom

    dadj = jnp.sum(adj * eyeM, axis=1, keepdims=True)
    dadjr = jnp.sum(adjr * eyeM, axis=1, keepdims=True)
    diag_term = outer(dadj, dadjr) * cosm

    f_r = lax.broadcasted_iota(jnp.int32, (N * N, N), 0)
    f_c = lax.broadcasted_iota(jnp.int32, (N * N, N), 1)
    RowSel = ((f_r // N) == f_c).astype(F32)
    ModMsk = ((f_r % N) == f_c).astype(F32)
    adj_col = jnp.sum(dot(RowSel, adj) * ModMsk,
                      axis=1, keepdims=True)

    g_r = lax.broadcasted_iota(jnp.int32, (N, N * N), 0)
    g_c = lax.broadcasted_iota(jnp.int32, (N, N * N), 1)
    ColSel = (g_r == (g_c % N)).astype(F32)
    DivMsk = (g_r == (g_c // N)).astype(F32)
    adjr_row = jnp.sum(dot(adjr, ColSel) * DivMsk,
                       axis=0, keepdims=True)
    base_S = jnp.abs(adj_col - adjr_row)

    vR = lax.broadcasted_iota(jnp.int32, (N * N, 1), 0)
    vC = lax.broadcasted_iota(jnp.int32, (1, N * N), 1)
    eyeR = (vR // N) == (vR % N)
    eyeC = (vC // N) == (vC % N)
    offmask = ((~eyeR) & (~eyeC)).astype(F32)

    dt_c = jnp.concatenate([diag_term] * N, axis=1)
    dt_tile = jnp.concatenate([dt_c] * N, axis=0)
    S = jnp.where(eyeR & eyeC, dt_tile, base_S * offmask)

    rm_r = lax.broadcasted_iota(jnp.int32, (N, N * N), 0)
    rm_c = lax.broadcasted_iota(jnp.int32, (N, N * N), 1)
    Rm = (((rm_c // N) == rm_r) & ((rm_c % N) != rm_r)).astype(F32)

    def mpm_core(x):
        xcols = jnp.concatenate([x] * N, axis=0)
        pmax = jnp.concatenate(
            [jnp.max(S[:, a * N:(a + 1) * N] * xcols, axis=1, keepdims=True)
             for a in range(N)], axis=1)
        neigh = dot(Rm, pmax)
        return x * diag_term + neigh

    def mpm_outer(_, x):
        x = lax.fori_loop(0, 9, lambda __, v: mpm_core(v), x)
        x = mpm_core(x)
        return x / jnp.sqrt(jnp.sum(x * x))

    x0 = jnp.full((N, N), 1.0 / N, F32)
    x = lax.fori_loop(0, 5, mpm_outer, x0)

    loss_kl = -0.5 * jnp.sum(1.0 + z_ls - z_mu * z_mu - jnp.exp(z_ls)) / (N * N)
    diff_e = edges_total - ef_all
    loss_edge = jnp.sum(diff_e * diff_e) / (EM * 4)
    diff_n = node_recon - gh
    loss_node = jnp.sum(diff_n * diff_n) / HH

    pclip = jnp.clip(low, 1e-7, 1.0 - 1e-7)
    logp = jnp.log(pclip)
    log1mp = jnp.log(1.0 - pclip)

    pad7 = jnp.zeros((N, 16 - N), F32)
    x_ref[...] = jnp.concatenate([x, pad7], axis=1)
    adj_ref16[...] = jnp.concatenate([adj, pad7], axis=1)
    logp_ref[...] = jnp.concatenate([logp, pad7], axis=1)
    log1mp_ref[...] = jnp.concatenate([log1mp, pad7], axis=1)
    part_ref[...] = jnp.broadcast_to(loss_kl + loss_edge + loss_node, (1, 16))


_SC_MESH = plsc.VectorSubcoreMesh(core_axis_name="c", subcore_axis_name="s")


@functools.partial(
    pl.kernel, mesh=_SC_MESH,
    compiler_params=pltpu.CompilerParams(use_tc_tiling_on_sc=False,
                                         needs_layout_passes=False),
    out_type=jax.ShapeDtypeStruct((1, 16), jnp.float32),
    scratch_types=[
        pltpu.VMEM((N, 16), jnp.float32),
        pltpu.VMEM((N * 16,), jnp.float32),
        pltpu.VMEM((N, 16), jnp.float32),
        pltpu.VMEM((N, 16), jnp.float32),
        pltpu.VMEM((1, 16), jnp.float32),
        pltpu.VMEM((512,), jnp.float32),
        pltpu.VMEM((512,), jnp.float32),
        pltpu.VMEM((9 * 512,), jnp.int32),
        pltpu.VMEM((1, 16), jnp.float32),
    ])
def _sc_assign(x_hbm, adj_hbm, logp_hbm, log1mp_hbm, part_hbm, out_hbm,
               x_v, adj_v, logp_v, log1mp_v, part_v, g_a, g_b, am_v,
               res_v):
    @pl.when((lax.axis_index("c") == 0) & (lax.axis_index("s") == 0))
    def _():
        pltpu.sync_copy(x_hbm, x_v)
        pltpu.sync_copy(adj_hbm, adj_v)
        pltpu.sync_copy(logp_hbm, logp_v)
        pltpu.sync_copy(log1mp_hbm, log1mp_v)
        pltpu.sync_copy(part_hbm, part_v)
        iota16 = lax.iota(jnp.int32, 16)
        neg = jnp.full((16,), NEG, jnp.float32)
        zero16 = jnp.zeros((16,), jnp.float32)

        def zinit(k, c):
            g_a[pl.ds(k * 16, 16)] = zero16
            return c
        lax.fori_loop(0, 32, zinit, 0)

        bufs = [g_a, g_b]
        for i in range(N - 1, -1, -1):
            gp = bufs[(8 - i) % 2]
            gn = bufs[(9 - i) % 2]

            xrow = x_v[i, :]

            def round_body(k, i=i, gp=gp, gn=gn, xrow=xrow):
                base = k * 16
                svec = base + iota16
                m = neg
                am = jnp.zeros((16,), jnp.int32)
                for j in range(N):
                    bit = ((svec >> j) & 1) == 1
                    gat = plsc.load_gather(gp, [svec - (1 << j)], mask=bit)
                    cand = jnp.where(bit, gat + xrow[j], neg)
                    am = jnp.where(cand > m, j, am)
                    m = jnp.maximum(m, cand)
                gn[pl.ds(base, 16)] = m
                am_v[pl.ds(i * 512 + base, 16)] = am
            plsc.parallel_loop(0, 32, unroll=4)(round_body)

        scur = jnp.full((16,), 511, jnp.int32)
        ind = jnp.zeros((16,), jnp.int32)
        jp_list = []
        for i in range(N):
            jp = plsc.load_gather(am_v, [i * 512 + scur])
            jp_list.append(jp)
            ind = ind + jnp.where(iota16 == jp, i, 0)
            scur = scur - (jnp.int32(1) << jp)

        acc = jnp.zeros((16,), jnp.float32)
        for r in range(N):
            ind_r = jnp.zeros((16,), jnp.int32)
            for i in range(N):
                ind_r = ind_r + jnp.where(jp_list[i] == r, i, 0)
            a_row = plsc.load_gather(adj_v, [ind_r * 16 + ind])
            lp = logp_v[r, :]
            l1 = log1mp_v[r, :]
            term = a_row * lp + (1.0 - a_row) * l1
            mask = (iota16 >= r) & (iota16 < N)
            acc = acc + jnp.where(mask, term, 0.0)
        s16 = jnp.broadcast_to(jnp.sum(acc), (16,))
        total = part_v[0, :] - s16 / jnp.full((16,), float(ODIM), jnp.float32)
        res_v[0, :] = total
        pltpu.sync_copy(res_v, out_hbm)


def kernel(adj, edges_features, nodes_features, W_mu, b_mu, W_ls, b_ls,
           W_d1, b_d1, W_d2, b_d2, W_nd, b_nd, W_ed, b_ed, eps):
    adj0 = adj[0]
    ef = edges_features[0]
    gh = nodes_features.reshape(1, HH)
    x16, adj16, logp16, log1mp16, part = pl.pallas_call(
        _body1,
        out_shape=[jax.ShapeDtypeStruct((N, 16), jnp.float32),
                   jax.ShapeDtypeStruct((N, 16), jnp.float32),
                   jax.ShapeDtypeStruct((N, 16), jnp.float32),
                   jax.ShapeDtypeStruct((N, 16), jnp.float32),
                   jax.ShapeDtypeStruct((1, 16), jnp.float32)],
    )(adj0, ef, gh,
      W_mu, b_mu.reshape(1, -1), W_ls, b_ls.reshape(1, -1),
      W_d1, b_d1.reshape(1, -1), W_d2, b_d2.reshape(1, -1),
      W_nd, b_nd.reshape(1, -1), W_ed, b_ed.reshape(1, -1),
      eps.reshape(1, -1))

    res = _sc_assign(x16, adj16.reshape(N * 16), logp16, log1mp16, part)
    return res[0, 0]
